# Initial kernel scaffold; baseline (speedup 1.0000x reference)
import jax
import jax.numpy as jnp
from jax.experimental import pallas as pl


def kernel(x):
    raise NotImplementedError("write your pallas kernel here")


# speedup vs baseline: 27.2256x; 27.2256x over previous
import functools

import jax
import jax.numpy as jnp
from jax import lax
from jax.experimental import pallas as pl
from jax.experimental.pallas import tpu as pltpu
from jax.experimental.pallas import tpu_sc as plsc

NC = 2
NS = 16
NW = NC * NS
L = 16

NCOPY = 2
CHUNK = 8000
UNROLL = 10


def _make_hist_kernel(n, nbins, shift, match_shift):
    per_w = n // NW
    assert per_w * NW == n and per_w % CHUNK == 0
    n_chunks = per_w // CHUNK
    assert n_chunks % 2 == 0
    nvreg = CHUNK // L
    assert nvreg % UNROLL == 0

    mesh = plsc.VectorSubcoreMesh(core_axis_name="c", subcore_axis_name="s")

    scratch = [
        pltpu.VMEM((CHUNK,), jnp.float32),
        pltpu.VMEM((CHUNK,), jnp.float32),
        pltpu.VMEM((nbins * NCOPY * L,), jnp.int32),
        pltpu.VMEM((nbins * L,), jnp.int32),
        pltpu.SemaphoreType.DMA,
        pltpu.SemaphoreType.DMA,
    ]
    if match_shift is not None:
        scratch.append(pltpu.VMEM((L,), jnp.int32))

    def body(*args):
        if match_shift is not None:
            x_hbm, pref_hbm, out_hbm, buf0, buf1, hist, red, sem_a, sem_b, pref_v = args
        else:
            x_hbm, out_hbm, buf0, buf1, hist, red, sem_a, sem_b = args
        sems = (sem_a, sem_b)
        bufs = (buf0, buf1)

        wid = lax.axis_index("s") * NC + lax.axis_index("c")
        base = wid * per_w

        zeros = jnp.zeros((L,), jnp.int32)

        def zero_body(i, _):
            hist[pl.ds(i * L, L)] = zeros
            return 0

        lax.fori_loop(0, nbins * NCOPY, zero_body, 0)

        if match_shift is not None:
            pltpu.sync_copy(pref_hbm, pref_v)
            pref_u = lax.bitcast_convert_type(pref_v[...], jnp.uint32)

        lane = lax.iota(jnp.int32, L)
        ones = jnp.ones((L,), jnp.int32)

        def start_dma(ci, b):
            return pltpu.async_copy(
                x_hbm.at[pl.ds(base + ci * CHUNK, CHUNK)], bufs[b], sems[b])

        start_dma(0, 0)
        start_dma(1, 1)

        def process(bufb, off, par):
            xv = bufb[pl.ds(off, L)]
            ui = lax.bitcast_convert_type(xv, jnp.int32)
            m = lax.shift_right_arithmetic(ui, 31)
            key_i = lax.bitwise_xor(ui, lax.bitwise_or(m, jnp.int32(-(2 ** 31))))
            key = lax.bitcast_convert_type(key_i, jnp.uint32)
            binv = lax.shift_right_logical(key, jnp.uint32(shift)) & jnp.uint32(nbins - 1)
            idx = (lax.bitcast_convert_type(binv, jnp.int32) * (NCOPY * L)
                   + (par * L + lane))
            if match_shift is not None:
                match = lax.shift_right_logical(key, jnp.uint32(match_shift)) == pref_u
                plsc.addupdate_scatter(hist, [idx], ones, mask=match)
            else:
                plsc.addupdate_scatter(hist, [idx], ones)

        def outer(ci0, _):
            for b in range(2):
                ci = 2 * ci0 + b
                pltpu.make_async_copy(
                    x_hbm.at[pl.ds(base + ci * CHUNK, CHUNK)], bufs[b],
                    sems[b]).wait()
                bufb = bufs[b]

                def vbody(vi, _):
                    off0 = vi * (L * UNROLL)
                    for u in range(UNROLL):
                        process(bufb, off0 + u * L, u % NCOPY)
                    return 0

                lax.fori_loop(0, nvreg // UNROLL, vbody, 0)

                @pl.when(ci + 2 < n_chunks)
                def _():
                    start_dma(ci + 2, b)
            return 0

        lax.fori_loop(0, n_chunks // 2, outer, 0, unroll=False)

        def red_body(g, _):
            acc = hist[pl.ds(g * (NCOPY * L), L)]
            for c in range(1, NCOPY):
                acc = acc + hist[pl.ds(g * (NCOPY * L) + c * L, L)]
            red[pl.ds(g * L, L)] = acc
            return 0

        lax.fori_loop(0, nbins, red_body, 0)
        pltpu.sync_copy(red, out_hbm.at[wid])

    out_type = jax.ShapeDtypeStruct((NW, nbins * L), jnp.int32)
    return pl.kernel(
        body, out_type=out_type, mesh=mesh, scratch_types=scratch,
        compiler_params=pltpu.CompilerParams(needs_layout_passes=False))


def _select(hists, nbins, k):
    h = jnp.sum(hists.reshape(NW, nbins, L), axis=(0, 2))
    c = jnp.cumsum(h[::-1])[::-1]
    t = jnp.sum((c >= k).astype(jnp.int32)) - 1
    cgt = c[t] - h[t]
    return t, k - cgt


def _mask_body(x_ref, thr_ref, o_ref):
    t = thr_ref[0]
    xv = x_ref[...]
    o_ref[...] = jnp.maximum(xv, 0.0) * (xv >= t).astype(jnp.float32)


def _mask_call(x, thr):
    B, D = x.shape
    bb, bd = 128, 12800
    grid = (B // bb, pl.cdiv(D, bd))
    return pl.pallas_call(
        _mask_body,
        grid=grid,
        in_specs=[
            pl.BlockSpec((bb, bd), lambda i, j: (i, j)),
            pl.BlockSpec(memory_space=pltpu.SMEM),
        ],
        out_specs=pl.BlockSpec((bb, bd), lambda i, j: (i, j)),
        out_shape=jax.ShapeDtypeStruct((B, D), jnp.float32),
    )(x, thr)


def kernel(x):
    B, D = x.shape
    n = B * D
    k_total = min(64 * B, n)

    xf = x.reshape(-1)

    hist_p1 = _make_hist_kernel(n, nbins=2048, shift=21, match_shift=None)
    hist_p2 = _make_hist_kernel(n, nbins=2048, shift=10, match_shift=21)
    hist_p3 = _make_hist_kernel(n, nbins=1024, shift=0, match_shift=10)

    h1 = hist_p1(xf)
    t1, k1 = _select(h1, 2048, jnp.int32(k_total))

    pref1 = jnp.broadcast_to(t1, (L,)).astype(jnp.int32)
    h2 = hist_p2(xf, pref1)
    t2, k2 = _select(h2, 2048, k1)

    p22 = (t1 << 11) | t2
    pref2 = jnp.broadcast_to(p22, (L,)).astype(jnp.int32)
    h3 = hist_p3(xf, pref2)
    t3, _ = _select(h3, 1024, k2)

    key = ((p22 << 10) | t3).astype(jnp.uint32)
    bits = jnp.where(key >= jnp.uint32(0x80000000),
                     key ^ jnp.uint32(0x80000000),
                     ~key)
    thr = lax.bitcast_convert_type(bits, jnp.float32).reshape((1,))

    return _mask_call(x, thr)


# speedup vs baseline: 73.5764x; 2.7025x over previous
import functools

import jax
import jax.numpy as jnp
from jax import lax
from jax.experimental import pallas as pl
from jax.experimental.pallas import tpu as pltpu
from jax.experimental.pallas import tpu_sc as plsc

NC = 2
NS = 16
NW = NC * NS
L = 16

NCOPY = 2
CHUNK = 8000
UNROLL = 10


def _make_hist_kernel(n, nbins, shift, match_shift):
    per_w = n // NW
    assert per_w * NW == n and per_w % CHUNK == 0
    n_chunks = per_w // CHUNK
    assert n_chunks % 2 == 0
    nvreg = CHUNK // L
    assert nvreg % UNROLL == 0

    mesh = plsc.VectorSubcoreMesh(core_axis_name="c", subcore_axis_name="s")

    scratch = [
        pltpu.VMEM((CHUNK,), jnp.float32),
        pltpu.VMEM((CHUNK,), jnp.float32),
        pltpu.VMEM((nbins * NCOPY * L,), jnp.int32),
        pltpu.VMEM((nbins * L,), jnp.int32),
        pltpu.SemaphoreType.DMA,
        pltpu.SemaphoreType.DMA,
    ]
    if match_shift is not None:
        scratch.append(pltpu.VMEM((L,), jnp.int32))

    def body(*args):
        if match_shift is not None:
            x_hbm, pref_hbm, out_hbm, buf0, buf1, hist, red, sem_a, sem_b, pref_v = args
        else:
            x_hbm, out_hbm, buf0, buf1, hist, red, sem_a, sem_b = args
        sems = (sem_a, sem_b)
        bufs = (buf0, buf1)

        wid = lax.axis_index("s") * NC + lax.axis_index("c")
        base = wid * per_w

        zeros = jnp.zeros((L,), jnp.int32)

        def zero_body(i, _):
            hist[pl.ds(i * L, L)] = zeros
            return 0

        lax.fori_loop(0, nbins * NCOPY, zero_body, 0)

        if match_shift is not None:
            pltpu.sync_copy(pref_hbm, pref_v)
            pref_u = lax.bitcast_convert_type(pref_v[...], jnp.uint32)

        lane = lax.iota(jnp.int32, L)
        ones = jnp.ones((L,), jnp.int32)

        def start_dma(ci, b):
            return pltpu.async_copy(
                x_hbm.at[pl.ds(base + ci * CHUNK, CHUNK)], bufs[b], sems[b])

        start_dma(0, 0)
        start_dma(1, 1)

        def process(bufb, off, par):
            xv = bufb[pl.ds(off, L)]
            ui = lax.bitcast_convert_type(xv, jnp.int32)
            m = lax.shift_right_arithmetic(ui, 31)
            key_i = lax.bitwise_xor(ui, lax.bitwise_or(m, jnp.int32(-(2 ** 31))))
            key = lax.bitcast_convert_type(key_i, jnp.uint32)
            binv = lax.shift_right_logical(key, jnp.uint32(shift))
            if (nbins << shift) < 2 ** 32:
                binv = binv & jnp.uint32(nbins - 1)
            idx = (lax.bitcast_convert_type(binv, jnp.int32) * (NCOPY * L)
                   + (par * L + lane))
            if match_shift is not None:
                match = lax.shift_right_logical(key, jnp.uint32(match_shift)) == pref_u
                plsc.addupdate_scatter(hist, [idx], ones, mask=match)
            else:
                plsc.addupdate_scatter(hist, [idx], ones)

        def outer(ci0, _):
            for b in range(2):
                ci = 2 * ci0 + b
                pltpu.make_async_copy(
                    x_hbm.at[pl.ds(base + ci * CHUNK, CHUNK)], bufs[b],
                    sems[b]).wait()
                bufb = bufs[b]

                @plsc.parallel_loop(0, nvreg, step=NCOPY, unroll=UNROLL // NCOPY)
                def _(vi):
                    for u in range(NCOPY):
                        process(bufb, (vi + u) * L, u)

                @pl.when(ci + 2 < n_chunks)
                def _():
                    start_dma(ci + 2, b)
            return 0

        lax.fori_loop(0, n_chunks // 2, outer, 0, unroll=False)

        def red_body(g, _):
            acc = hist[pl.ds(g * (NCOPY * L), L)]
            for c in range(1, NCOPY):
                acc = acc + hist[pl.ds(g * (NCOPY * L) + c * L, L)]
            red[pl.ds(g * L, L)] = acc
            return 0

        lax.fori_loop(0, nbins, red_body, 0)
        pltpu.sync_copy(red, out_hbm.at[wid])

    out_type = jax.ShapeDtypeStruct((NW, nbins * L), jnp.int32)
    return pl.kernel(
        body, out_type=out_type, mesh=mesh, scratch_types=scratch,
        compiler_params=pltpu.CompilerParams(needs_layout_passes=False))


def _select(hists, nbins, k):
    h = jnp.sum(hists.reshape(NW, nbins, L), axis=(0, 2))
    c = jnp.cumsum(h[::-1])[::-1]
    t = jnp.sum((c >= k).astype(jnp.int32)) - 1
    cgt = c[t] - h[t]
    return t, k - cgt


def _mask_body(x_ref, thr_ref, o_ref):
    t = thr_ref[0]
    xv = x_ref[...]
    o_ref[...] = jnp.maximum(xv, 0.0) * (xv >= t).astype(jnp.float32)


def _mask_call(x, thr):
    B, D = x.shape
    bb, bd = 128, 12800
    grid = (B // bb, pl.cdiv(D, bd))
    return pl.pallas_call(
        _mask_body,
        grid=grid,
        in_specs=[
            pl.BlockSpec((bb, bd), lambda i, j: (i, j)),
            pl.BlockSpec(memory_space=pltpu.SMEM),
        ],
        out_specs=pl.BlockSpec((bb, bd), lambda i, j: (i, j)),
        out_shape=jax.ShapeDtypeStruct((B, D), jnp.float32),
    )(x, thr)


def kernel(x):
    B, D = x.shape
    n = B * D
    k_total = min(64 * B, n)

    xf = x.reshape(-1)

    hist_p1 = _make_hist_kernel(n, nbins=2048, shift=21, match_shift=None)
    hist_p2 = _make_hist_kernel(n, nbins=2048, shift=10, match_shift=21)
    hist_p3 = _make_hist_kernel(n, nbins=1024, shift=0, match_shift=10)

    h1 = hist_p1(xf)
    t1, k1 = _select(h1, 2048, jnp.int32(k_total))

    pref1 = jnp.broadcast_to(t1, (L,)).astype(jnp.int32)
    h2 = hist_p2(xf, pref1)
    t2, k2 = _select(h2, 2048, k1)

    p22 = (t1 << 11) | t2
    pref2 = jnp.broadcast_to(p22, (L,)).astype(jnp.int32)
    h3 = hist_p3(xf, pref2)
    t3, _ = _select(h3, 1024, k2)

    key = ((p22 << 10) | t3).astype(jnp.uint32)
    bits = jnp.where(key >= jnp.uint32(0x80000000),
                     key ^ jnp.uint32(0x80000000),
                     ~key)
    thr = lax.bitcast_convert_type(bits, jnp.float32).reshape((1,))

    return _mask_call(x, thr)


# speedup vs baseline: 77.1649x; 1.0488x over previous
import jax
import jax.numpy as jnp
from jax import lax
from jax.experimental import pallas as pl
from jax.experimental.pallas import tpu as pltpu
from jax.experimental.pallas import tpu_sc as plsc

NC = 2
NS = 16
NW = NC * NS
L = 16

NCOPY = 2
CHUNK = 12800
UNROLL = 10


def _make_hist_kernel(n, nbins, shift, match_shift):
    per_w = n // NW
    assert per_w * NW == n and per_w % CHUNK == 0
    n_chunks = per_w // CHUNK
    assert n_chunks % 2 == 0
    nvreg = CHUNK // L
    assert nvreg % UNROLL == 0

    mesh = plsc.VectorSubcoreMesh(core_axis_name="c", subcore_axis_name="s")

    scratch = [
        pltpu.VMEM((CHUNK,), jnp.float32),
        pltpu.VMEM((CHUNK,), jnp.float32),
        pltpu.VMEM((nbins * NCOPY * L,), jnp.int32),
        pltpu.VMEM((nbins * L,), jnp.int32),
        pltpu.SemaphoreType.DMA,
        pltpu.SemaphoreType.DMA,
    ]
    if match_shift is not None:
        scratch.append(pltpu.VMEM((L,), jnp.int32))

    def body(*args):
        if match_shift is not None:
            x_hbm, pref_hbm, out_hbm, buf0, buf1, hist, red, sem_a, sem_b, pref_v = args
        else:
            x_hbm, out_hbm, buf0, buf1, hist, red, sem_a, sem_b = args
        sems = (sem_a, sem_b)
        bufs = (buf0, buf1)

        wid = lax.axis_index("s") * NC + lax.axis_index("c")
        base = wid * per_w

        zeros = jnp.zeros((L,), jnp.int32)

        def zero_body(i, _):
            hist[pl.ds(i * L, L)] = zeros
            return 0

        lax.fori_loop(0, nbins * NCOPY, zero_body, 0)

        if match_shift is not None:
            pltpu.sync_copy(pref_hbm, pref_v)
            pref_u = lax.bitcast_convert_type(pref_v[...], jnp.uint32)

        lane = lax.iota(jnp.int32, L)
        ones = jnp.ones((L,), jnp.int32)

        def start_dma(ci, b):
            return pltpu.async_copy(
                x_hbm.at[pl.ds(base + ci * CHUNK, CHUNK)], bufs[b], sems[b])

        start_dma(0, 0)
        start_dma(1, 1)

        def process(bufb, off, par):
            xv = bufb[pl.ds(off, L)]
            ui = lax.bitcast_convert_type(xv, jnp.int32)
            m = lax.shift_right_arithmetic(ui, 31)
            key_i = lax.bitwise_xor(ui, lax.bitwise_or(m, jnp.int32(-(2 ** 31))))
            key = lax.bitcast_convert_type(key_i, jnp.uint32)
            binv = lax.shift_right_logical(key, jnp.uint32(shift))
            if (nbins << shift) < 2 ** 32:
                binv = binv & jnp.uint32(nbins - 1)
            idx = (lax.bitcast_convert_type(binv, jnp.int32) * (NCOPY * L)
                   + (par * L + lane))
            if match_shift is not None:
                match = lax.shift_right_logical(key, jnp.uint32(match_shift)) == pref_u
                plsc.addupdate_scatter(hist, [idx], ones, mask=match)
            else:
                plsc.addupdate_scatter(hist, [idx], ones)

        def outer(ci0, _):
            for b in range(2):
                ci = 2 * ci0 + b
                pltpu.make_async_copy(
                    x_hbm.at[pl.ds(base + ci * CHUNK, CHUNK)], bufs[b],
                    sems[b]).wait()
                bufb = bufs[b]

                @plsc.parallel_loop(0, nvreg, step=NCOPY, unroll=UNROLL // NCOPY)
                def _(vi):
                    for u in range(NCOPY):
                        process(bufb, (vi + u) * L, u)

                @pl.when(ci + 2 < n_chunks)
                def _():
                    start_dma(ci + 2, b)
            return 0

        lax.fori_loop(0, n_chunks // 2, outer, 0, unroll=False)

        def red_body(g, _):
            acc = hist[pl.ds(g * (NCOPY * L), L)]
            for c in range(1, NCOPY):
                acc = acc + hist[pl.ds(g * (NCOPY * L) + c * L, L)]
            red[pl.ds(g * L, L)] = acc
            return 0

        lax.fori_loop(0, nbins, red_body, 0)
        pltpu.sync_copy(red, out_hbm.at[wid])

    out_type = jax.ShapeDtypeStruct((NW, nbins * L), jnp.int32)
    return pl.kernel(
        body, out_type=out_type, mesh=mesh, scratch_types=scratch,
        compiler_params=pltpu.CompilerParams(needs_layout_passes=False))


def _select(hists, nbins, k):
    h = jnp.sum(hists.reshape(NW, nbins, L), axis=(0, 2))
    c = jnp.cumsum(h[::-1])[::-1]
    t = jnp.sum((c >= k).astype(jnp.int32)) - 1
    cgt = c[t] - h[t]
    return t, k - cgt


def _mask_body(x_ref, thr_ref, o_ref):
    t = thr_ref[0]
    xv = x_ref[...]
    o_ref[...] = jnp.maximum(xv, 0.0) * (xv >= t).astype(jnp.float32)


def _mask_call(x, thr):
    B, D = x.shape
    bb, bd = 128, 12800
    grid = (B // bb, pl.cdiv(D, bd))
    return pl.pallas_call(
        _mask_body,
        grid=grid,
        in_specs=[
            pl.BlockSpec((bb, bd), lambda i, j: (i, j)),
            pl.BlockSpec(memory_space=pltpu.SMEM),
        ],
        out_specs=pl.BlockSpec((bb, bd), lambda i, j: (i, j)),
        out_shape=jax.ShapeDtypeStruct((B, D), jnp.float32),
    )(x, thr)


def kernel(x):
    B, D = x.shape
    n = B * D
    k_total = min(64 * B, n)

    xf = x.reshape(-1)

    hist_p1 = _make_hist_kernel(n, nbins=2048, shift=21, match_shift=None)
    hist_p2 = _make_hist_kernel(n, nbins=2048, shift=10, match_shift=21)
    hist_p3 = _make_hist_kernel(n, nbins=1024, shift=0, match_shift=10)

    h1 = hist_p1(xf)
    t1, k1 = _select(h1, 2048, jnp.int32(k_total))

    pref1 = jnp.broadcast_to(t1, (L,)).astype(jnp.int32)
    h2 = hist_p2(xf, pref1)
    t2, k2 = _select(h2, 2048, k1)

    p22 = (t1 << 11) | t2
    pref2 = jnp.broadcast_to(p22, (L,)).astype(jnp.int32)
    h3 = hist_p3(xf, pref2)
    t3, _ = _select(h3, 1024, k2)

    key = ((p22 << 10) | t3).astype(jnp.uint32)
    bits = jnp.where(key >= jnp.uint32(0x80000000),
                     key ^ jnp.uint32(0x80000000),
                     ~key)
    thr = lax.bitcast_convert_type(bits, jnp.float32).reshape((1,))

    return _mask_call(x, thr)
